# baseline (speedup 1.0000x reference)
import jax
import jax.numpy as jnp
from jax.experimental import pallas as pl
from jax.experimental.pallas import tpu as pltpu

TOKENS = 8192
D_MODEL = 4096
UNITS = 64
TEMP = 0.5

BLOCK_M = 512


def _router_kernel(xa_ref, xb_ref, w_ref, b_ref, o_ref):
    w = w_ref[...]
    b = b_ref[...]
    za = jnp.dot(xa_ref[...].astype(jnp.bfloat16), w,
                 preferred_element_type=jnp.float32)
    o_ref[:BLOCK_M, :] = 0.5 * jnp.tanh(za + b) + 0.5
    zb = jnp.dot(xb_ref[...].astype(jnp.bfloat16), w,
                 preferred_element_type=jnp.float32)
    o_ref[BLOCK_M:, :] = 0.5 * jnp.tanh(zb + b) + 0.5


def kernel(inputs, proj, logit_bias):
    s = 0.5 / (TEMP + 1e-08)
    w2 = (proj * s).astype(jnp.bfloat16)
    b2 = (logit_bias * s).reshape(1, UNITS)
    grid = (TOKENS // (2 * BLOCK_M),)
    return pl.pallas_call(
        _router_kernel,
        grid=grid,
        in_specs=[
            pl.BlockSpec((BLOCK_M, D_MODEL), lambda i: (2 * i, 0)),
            pl.BlockSpec((BLOCK_M, D_MODEL), lambda i: (2 * i + 1, 0)),
            pl.BlockSpec((D_MODEL, UNITS), lambda i: (0, 0)),
            pl.BlockSpec((1, UNITS), lambda i: (0, 0)),
        ],
        out_specs=pl.BlockSpec((2 * BLOCK_M, UNITS), lambda i: (i, 0)),
        out_shape=jax.ShapeDtypeStruct((TOKENS, UNITS), jnp.float32),
        compiler_params=pltpu.CompilerParams(
            dimension_semantics=("parallel",),
            allow_input_fusion=[False, False, True, True],
        ),
    )(inputs, inputs, w2, b2)


# speedup vs baseline: 1.1414x; 1.1414x over previous
import jax
import jax.numpy as jnp
from jax.experimental import pallas as pl
from jax.experimental.pallas import tpu as pltpu

TOKENS = 8192
D_MODEL = 4096
UNITS = 64
TEMP = 0.5

BLOCK_M = 512


def _router_kernel(x_ref, w_ref, b_ref, o_ref):
    x = x_ref[...].astype(jnp.bfloat16)
    w = w_ref[...]
    z = jnp.dot(x, w, preferred_element_type=jnp.float32)
    o_ref[...] = 0.5 * jnp.tanh(z + b_ref[...]) + 0.5


def kernel(inputs, proj, logit_bias):
    s = 0.5 / (TEMP + 1e-08)
    w2 = (proj * s).astype(jnp.bfloat16)
    b2 = (logit_bias * s).reshape(1, UNITS)
    grid = (TOKENS // BLOCK_M,)
    return pl.pallas_call(
        _router_kernel,
        grid=grid,
        in_specs=[
            pl.BlockSpec((BLOCK_M, D_MODEL), lambda i: (i, 0)),
            pl.BlockSpec((D_MODEL, UNITS), lambda i: (0, 0)),
            pl.BlockSpec((1, UNITS), lambda i: (0, 0)),
        ],
        out_specs=pl.BlockSpec((BLOCK_M, UNITS), lambda i: (i, 0)),
        out_shape=jax.ShapeDtypeStruct((TOKENS, UNITS), jnp.float32),
        compiler_params=pltpu.CompilerParams(
            dimension_semantics=("parallel",),
            allow_input_fusion=[False, True, True],
        ),
    )(inputs, w2, b2)
